# baseline (speedup 1.0000x reference)
import functools

import jax
import jax.numpy as jnp
from jax import lax
from jax.experimental import pallas as pl
from jax.experimental.pallas import tpu as pltpu
from jax.experimental.pallas import tpu_sc as plsc

BATCH = 16384
DIM = 64
NC = 2
NS = 16
NW = NC * NS
B_PER_W = BATCH // NW
CHUNK = 128
N_CHUNKS = B_PER_W // CHUNK

MLP_BLOCK = 2048


def _sc_gather_pair():
    mesh = plsc.VectorSubcoreMesh(core_axis_name="c", subcore_axis_name="s")

    @functools.partial(
        pl.kernel,
        mesh=mesh,
        out_type=[
            jax.ShapeDtypeStruct((BATCH, 2 * DIM), jnp.float32),
            jax.ShapeDtypeStruct((BATCH, 2 * DIM), jnp.float32),
        ],
        scratch_types=[
            pltpu.VMEM((B_PER_W,), jnp.int32),
            pltpu.VMEM((B_PER_W,), jnp.int32),
            pltpu.VMEM((B_PER_W, 2 * DIM), jnp.float32),
            pltpu.SemaphoreType.DMA,
        ],
    )
    def gather_k(uidx_hbm, iidx_hbm, ctab_hbm, uout_hbm, vout_hbm,
                 uidx_v, iidx_v, rows_v, sem):
        wid = lax.axis_index("s") * NC + lax.axis_index("c")
        base = wid * B_PER_W
        pltpu.sync_copy(uidx_hbm.at[pl.ds(base, B_PER_W)], uidx_v)
        pltpu.sync_copy(iidx_hbm.at[pl.ds(base, B_PER_W)], iidx_v)
        for idx_v, out_hbm in ((uidx_v, uout_hbm), (iidx_v, vout_hbm)):
            copies = []
            for j in range(N_CHUNKS):
                sl = pl.ds(j * CHUNK, CHUNK)
                copies.append(pltpu.async_copy(
                    ctab_hbm.at[idx_v.at[sl]], rows_v.at[sl], sem))
            for c in copies:
                c.wait()
            pltpu.sync_copy(rows_v, out_hbm.at[pl.ds(base, B_PER_W)])

    return gather_k


CONCAT_BLOCK = 8192


def _concat_body(u_ref, v_ref, o_ref):
    o_ref[:, :DIM] = u_ref[...]
    o_ref[:, DIM:] = v_ref[...]


def _concat_tables(user_table, item_table):
    n = user_table.shape[0]
    nb = (n + CONCAT_BLOCK - 1) // CONCAT_BLOCK
    return pl.pallas_call(
        _concat_body,
        grid=(nb,),
        in_specs=[pl.BlockSpec((CONCAT_BLOCK, DIM), lambda i: (i, 0)),
                  pl.BlockSpec((CONCAT_BLOCK, DIM), lambda i: (i, 0))],
        out_specs=pl.BlockSpec((CONCAT_BLOCK, 2 * DIM), lambda i: (i, 0)),
        out_shape=jax.ShapeDtypeStruct((n, 2 * DIM), jnp.float32),
        compiler_params=pltpu.CompilerParams(
            dimension_semantics=("arbitrary",)),
    )(user_table, item_table)


def _ln(x, g, b):
    m = jnp.mean(x, axis=-1, keepdims=True)
    v = jnp.mean((x - m) ** 2, axis=-1, keepdims=True)
    return (x - m) / jnp.sqrt(v + 1e-5) * g + b


def _dot(a, b):
    return jnp.dot(a, b, preferred_element_type=jnp.float32)


def _mlp_body(u_ref, v_ref, w0_ref, b0_ref, g0_ref, be0_ref,
              w1_ref, b1_ref, g1_ref, be1_ref,
              w2_ref, b2_ref, g2_ref, be2_ref,
              wl_ref, bl_ref, o_ref):
    x = jnp.concatenate([u_ref[:, :DIM], v_ref[:, DIM:]], axis=-1)
    x = _dot(x, w0_ref[...]) + b0_ref[...]
    x = jax.nn.relu(_ln(x, g0_ref[...], be0_ref[...]))
    x = _dot(x, w1_ref[...]) + b1_ref[...]
    x = jax.nn.relu(_ln(x, g1_ref[...], be1_ref[...]))
    x = _dot(x, w2_ref[...]) + b2_ref[...]
    x = jax.nn.relu(_ln(x, g2_ref[...], be2_ref[...]))
    o_ref[...] = (_dot(x, wl_ref[...]) + bl_ref[0, 0]).reshape(o_ref.shape)


def _mlp_call(u, v, W0, b0, g0, be0, W1, b1, g1, be1,
              W2, b2, g2, be2, Wl, bl, interpret=False):
    nb = BATCH // MLP_BLOCK
    full = lambda shape: pl.BlockSpec(shape, lambda i: (0, 0))
    return pl.pallas_call(
        _mlp_body,
        grid=(nb,),
        in_specs=[
            pl.BlockSpec((MLP_BLOCK, 2 * DIM), lambda i: (i, 0)),
            pl.BlockSpec((MLP_BLOCK, 2 * DIM), lambda i: (i, 0)),
            full((128, 256)), full((1, 256)), full((1, 256)), full((1, 256)),
            full((256, 128)), full((1, 128)), full((1, 128)), full((1, 128)),
            full((128, 64)), full((1, 64)), full((1, 64)), full((1, 64)),
            full((64, 1)), full((1, 1)),
        ],
        out_specs=pl.BlockSpec((MLP_BLOCK,), lambda i: (i,)),
        out_shape=jax.ShapeDtypeStruct((BATCH,), jnp.float32),
        compiler_params=pltpu.CompilerParams(
            dimension_semantics=("arbitrary",)),
        interpret=interpret,
    )(u, v, W0, b0.reshape(1, -1), g0.reshape(1, -1), be0.reshape(1, -1),
      W1, b1.reshape(1, -1), g1.reshape(1, -1), be1.reshape(1, -1),
      W2, b2.reshape(1, -1), g2.reshape(1, -1), be2.reshape(1, -1),
      Wl, bl.reshape(1, 1))


def kernel(user_idx, item_idx, user_table, item_table,
           W0, b0, g0, be0, W1, b1, g1, be1, W2, b2, g2, be2, Wl, bl):
    ctab = _concat_tables(user_table, item_table)
    u, v = _sc_gather_pair()(user_idx, item_idx, ctab)
    return _mlp_call(u, v, W0, b0, g0, be0, W1, b1, g1, be1,
                     W2, b2, g2, be2, Wl, bl)


# speedup vs baseline: 1.1747x; 1.1747x over previous
import functools

import jax
import jax.numpy as jnp
from jax import lax
from jax.experimental import pallas as pl
from jax.experimental.pallas import tpu as pltpu
from jax.experimental.pallas import tpu_sc as plsc

BATCH = 16384
DIM = 64
NC = 2
NS = 16
NW = NC * NS
CHUNK = 128

N_SLICES = 2
SLICE = BATCH // N_SLICES
MLP_BLOCK = 2048


def _sc_gather_pair(batch):
    b_per_w = batch // NW
    n_chunks = b_per_w // CHUNK
    mesh = plsc.VectorSubcoreMesh(core_axis_name="c", subcore_axis_name="s")

    @functools.partial(
        pl.kernel,
        mesh=mesh,
        out_type=[
            jax.ShapeDtypeStruct((batch, 2 * DIM), jnp.float32),
            jax.ShapeDtypeStruct((batch, 2 * DIM), jnp.float32),
        ],
        scratch_types=[
            pltpu.VMEM((b_per_w,), jnp.int32),
            pltpu.VMEM((b_per_w,), jnp.int32),
            pltpu.VMEM((b_per_w, 2 * DIM), jnp.float32),
            pltpu.SemaphoreType.DMA,
        ],
    )
    def gather_k(uidx_hbm, iidx_hbm, ctab_hbm, uout_hbm, vout_hbm,
                 uidx_v, iidx_v, rows_v, sem):
        wid = lax.axis_index("s") * NC + lax.axis_index("c")
        base = wid * b_per_w
        pltpu.sync_copy(uidx_hbm.at[pl.ds(base, b_per_w)], uidx_v)
        pltpu.sync_copy(iidx_hbm.at[pl.ds(base, b_per_w)], iidx_v)
        for idx_v, out_hbm in ((uidx_v, uout_hbm), (iidx_v, vout_hbm)):
            copies = []
            for j in range(n_chunks):
                sl = pl.ds(j * CHUNK, CHUNK)
                copies.append(pltpu.async_copy(
                    ctab_hbm.at[idx_v.at[sl]], rows_v.at[sl], sem))
            for c in copies:
                c.wait()
            pltpu.sync_copy(rows_v, out_hbm.at[pl.ds(base, b_per_w)])

    return gather_k


def _ln(x, g, b):
    m = jnp.mean(x, axis=-1, keepdims=True)
    v = jnp.mean((x - m) ** 2, axis=-1, keepdims=True)
    return (x - m) / jnp.sqrt(v + 1e-5) * g + b


def _dot(a, b):
    return jnp.dot(a, b, preferred_element_type=jnp.float32)


def _mlp_body(u_ref, v_ref, w0_ref, b0_ref, g0_ref, be0_ref,
              w1_ref, b1_ref, g1_ref, be1_ref,
              w2_ref, b2_ref, g2_ref, be2_ref,
              wl_ref, bl_ref, o_ref):
    x = jnp.concatenate([u_ref[:, :DIM], v_ref[:, DIM:]], axis=-1)
    x = _dot(x, w0_ref[...]) + b0_ref[...]
    x = jax.nn.relu(_ln(x, g0_ref[...], be0_ref[...]))
    x = _dot(x, w1_ref[...]) + b1_ref[...]
    x = jax.nn.relu(_ln(x, g1_ref[...], be1_ref[...]))
    x = _dot(x, w2_ref[...]) + b2_ref[...]
    x = jax.nn.relu(_ln(x, g2_ref[...], be2_ref[...]))
    o_ref[...] = (_dot(x, wl_ref[...]) + bl_ref[0, 0]).reshape(o_ref.shape)


def _mlp_call(u, v, W0, b0, g0, be0, W1, b1, g1, be1,
              W2, b2, g2, be2, Wl, bl, interpret=False):
    batch = u.shape[0]
    nb = batch // MLP_BLOCK
    full = lambda shape: pl.BlockSpec(shape, lambda i: (0, 0))
    return pl.pallas_call(
        _mlp_body,
        grid=(nb,),
        in_specs=[
            pl.BlockSpec((MLP_BLOCK, 2 * DIM), lambda i: (i, 0)),
            pl.BlockSpec((MLP_BLOCK, 2 * DIM), lambda i: (i, 0)),
            full((128, 256)), full((1, 256)), full((1, 256)), full((1, 256)),
            full((256, 128)), full((1, 128)), full((1, 128)), full((1, 128)),
            full((128, 64)), full((1, 64)), full((1, 64)), full((1, 64)),
            full((64, 1)), full((1, 1)),
        ],
        out_specs=pl.BlockSpec((MLP_BLOCK,), lambda i: (i,)),
        out_shape=jax.ShapeDtypeStruct((batch,), jnp.float32),
        compiler_params=pltpu.CompilerParams(
            dimension_semantics=("arbitrary",)),
        interpret=interpret,
    )(u, v, W0, b0.reshape(1, -1), g0.reshape(1, -1), be0.reshape(1, -1),
      W1, b1.reshape(1, -1), g1.reshape(1, -1), be1.reshape(1, -1),
      W2, b2.reshape(1, -1), g2.reshape(1, -1), be2.reshape(1, -1),
      Wl, bl.reshape(1, 1))


def kernel(user_idx, item_idx, user_table, item_table,
           W0, b0, g0, be0, W1, b1, g1, be1, W2, b2, g2, be2, Wl, bl):
    ctab = jnp.concatenate([user_table, item_table], axis=1)
    gather = _sc_gather_pair(SLICE)
    outs = []
    for s in range(N_SLICES):
        sl = slice(s * SLICE, (s + 1) * SLICE)
        u, v = gather(user_idx[sl], item_idx[sl], ctab)
        outs.append(_mlp_call(u, v, W0, b0, g0, be0, W1, b1, g1, be1,
                              W2, b2, g2, be2, Wl, bl))
    return jnp.concatenate(outs)
